# baseline (speedup 1.0000x reference)
import jax
import jax.numpy as jnp
from jax.experimental import pallas as pl

C_BLK = 8
G_BLK = 400
N_GENES = 2000
N_EMB = 100


def _probe(emb_ref, out_ref):
    out_ref[:, :G_BLK] = emb_ref[:, :, 7]


@jax.jit
def kernel(cell_gene_embedding, gene_ix, bias1):
    n_cells = cell_gene_embedding.shape[0]
    grid = (n_cells // C_BLK, N_GENES // G_BLK)
    return pl.pallas_call(
        _probe,
        grid=grid,
        in_specs=[pl.BlockSpec((C_BLK, G_BLK, N_EMB), lambda i, j: (i, j, 0))],
        out_specs=pl.BlockSpec((C_BLK, N_GENES), lambda i, j: (i, 0)),
        out_shape=jax.ShapeDtypeStruct((n_cells, N_GENES), jnp.float32),
    )(cell_gene_embedding)


# speedup vs baseline: 1.1094x; 1.1094x over previous
import jax
import jax.numpy as jnp
from jax.experimental import pallas as pl
import jax.experimental.pallas.tpu as pltpu

C_BLK = 8
N_CELLS = 256
N_GENES = 2000
N_EMB = 100
N_BIAS = 128
DEPTH = 6
NBLKS = N_CELLS // C_BLK


def _mean_kernel(x_hbm, gix_ref, bias_ref, out_ref, buf, sem):
    i = pl.program_id(0)

    def copy(slot, blk):
        pltpu.make_async_copy(
            x_hbm.at[pl.ds(blk * C_BLK, C_BLK)],
            buf.at[slot],
            sem.at[slot],
        ).start()

    @pl.when(i == 0)
    def _():
        for d in range(DEPTH - 1):
            copy(d, d)

    nxt = i + DEPTH - 1

    @pl.when(nxt < NBLKS)
    def _():
        copy(jax.lax.rem(nxt, DEPTH), nxt)

    slot = jax.lax.rem(i, DEPTH)
    pltpu.make_async_copy(
        x_hbm.at[pl.ds(i * C_BLK, C_BLK)],
        buf.at[slot],
        sem.at[slot],
    ).wait()

    x = buf[slot]
    s = jnp.sum(x, axis=-1) * (1.0 / N_EMB)
    gix = gix_ref[...]
    bias = bias_ref[...]
    col = jax.lax.broadcasted_iota(jnp.int32, (N_GENES, N_BIAS), 1)
    onehot = gix[0][:, None] == col
    brow = jnp.sum(jnp.where(onehot, bias, 0.0), axis=1)[None, :]
    out_ref[...] = s + brow


@jax.jit
def kernel(cell_gene_embedding, gene_ix, bias1):
    gix2 = gene_ix.astype(jnp.int32).reshape(1, N_GENES)
    bias2 = bias1.reshape(1, N_BIAS)
    return pl.pallas_call(
        _mean_kernel,
        grid=(NBLKS,),
        in_specs=[
            pl.BlockSpec(memory_space=pltpu.MemorySpace.HBM),
            pl.BlockSpec((1, N_GENES), lambda i: (0, 0)),
            pl.BlockSpec((1, N_BIAS), lambda i: (0, 0)),
        ],
        out_specs=pl.BlockSpec((C_BLK, N_GENES), lambda i: (i, 0)),
        out_shape=jax.ShapeDtypeStruct((N_CELLS, N_GENES), jnp.float32),
        scratch_shapes=[
            pltpu.VMEM((DEPTH, C_BLK, N_GENES, N_EMB), jnp.float32),
            pltpu.SemaphoreType.DMA((DEPTH,)),
        ],
    )(cell_gene_embedding, gix2, bias2)


# speedup vs baseline: 1.6048x; 1.4465x over previous
import jax
import jax.numpy as jnp
from jax.experimental import pallas as pl
import jax.experimental.pallas.tpu as pltpu

C_BLK = 8
N_CELLS = 256
N_GENES = 2000
N_EMB = 100
N_BIAS = 128


def _mean_bias_kernel(emb_ref, gix_ref, bias_ref, out_ref):
    x = emb_ref[...]
    s = jnp.sum(x, axis=1) * (1.0 / N_EMB)
    gix = gix_ref[...]
    bias = bias_ref[...]
    col = jax.lax.broadcasted_iota(jnp.int32, (N_GENES, N_BIAS), 1)
    onehot = gix[0][:, None] == col
    brow = jnp.sum(jnp.where(onehot, bias, 0.0), axis=1)[None, :]
    out_ref[...] = s + brow


@jax.jit
def kernel(cell_gene_embedding, gene_ix, bias1):
    x_t = jnp.swapaxes(cell_gene_embedding, 1, 2)
    gix2 = gene_ix.astype(jnp.int32).reshape(1, N_GENES)
    bias2 = bias1.reshape(1, N_BIAS)
    return pl.pallas_call(
        _mean_bias_kernel,
        grid=(N_CELLS // C_BLK,),
        in_specs=[
            pl.BlockSpec((C_BLK, N_EMB, N_GENES), lambda i: (i, 0, 0)),
            pl.BlockSpec((1, N_GENES), lambda i: (0, 0)),
            pl.BlockSpec((1, N_BIAS), lambda i: (0, 0)),
        ],
        out_specs=pl.BlockSpec((C_BLK, N_GENES), lambda i: (i, 0)),
        out_shape=jax.ShapeDtypeStruct((N_CELLS, N_GENES), jnp.float32),
    )(x_t, gix2, bias2)


# speedup vs baseline: 1.6060x; 1.0008x over previous
import jax
import jax.numpy as jnp
from jax.experimental import pallas as pl
import jax.experimental.pallas.tpu as pltpu
from jax.experimental.pallas import tpu_sc as plsc

C_BLK = 8
N_CELLS = 256
N_GENES = 2000
N_EMB = 100
N_BIAS = 128
LANES = 16


def _bias_gather_sc(gix_hbm, bias_hbm, brow_hbm, gix_v, bias_v, brow_v):
    cid = jax.lax.axis_index("c")
    sid = jax.lax.axis_index("s")

    @pl.when(jnp.logical_and(cid == 0, sid == 0))
    def _():
        pltpu.sync_copy(gix_hbm, gix_v)
        pltpu.sync_copy(bias_hbm, bias_v)

        def body(i, carry):
            idx = gix_v[pl.ds(i * LANES, LANES)]
            brow_v[pl.ds(i * LANES, LANES)] = plsc.load_gather(bias_v, [idx])
            return carry

        jax.lax.fori_loop(0, N_GENES // LANES, body, 0)
        pltpu.sync_copy(brow_v, brow_hbm)


def _mean_kernel(emb_ref, brow_ref, out_ref):
    x = emb_ref[...]
    s = jnp.sum(x, axis=1) * (1.0 / N_EMB)
    out_ref[...] = s + brow_ref[...]


@jax.jit
def kernel(cell_gene_embedding, gene_ix, bias1):
    sc_gather = pl.kernel(
        _bias_gather_sc,
        mesh=plsc.VectorSubcoreMesh(core_axis_name="c", subcore_axis_name="s"),
        out_type=jax.ShapeDtypeStruct((N_GENES,), jnp.float32),
        scratch_types=[
            pltpu.VMEM((N_GENES,), jnp.int32),
            pltpu.VMEM((N_BIAS,), jnp.float32),
            pltpu.VMEM((N_GENES,), jnp.float32),
        ],
        compiler_params=pltpu.CompilerParams(needs_layout_passes=False),
    )
    brow = sc_gather(gene_ix.astype(jnp.int32), bias1)
    x_t = jnp.swapaxes(cell_gene_embedding, 1, 2)
    return pl.pallas_call(
        _mean_kernel,
        grid=(N_CELLS // C_BLK,),
        in_specs=[
            pl.BlockSpec((C_BLK, N_EMB, N_GENES), lambda i: (i, 0, 0)),
            pl.BlockSpec((1, N_GENES), lambda i: (0, 0)),
        ],
        out_specs=pl.BlockSpec((C_BLK, N_GENES), lambda i: (i, 0)),
        out_shape=jax.ShapeDtypeStruct((N_CELLS, N_GENES), jnp.float32),
    )(x_t, brow.reshape(1, N_GENES))
